# Initial kernel scaffold; baseline (speedup 1.0000x reference)
import jax
import jax.numpy as jnp
from jax.experimental import pallas as pl


def kernel(x, V, edge_index, n_id, emb_table, gin0_W1, gin0_b1, gin0_W2, gin0_b2, gin1_W1, gin1_b1, gin1_W2, gin1_b2, rho_W1, rho_b1, rho_W2, rho_b2, pe_W, pe_b, sage0_Wself, sage0_Wneigh, sage0_b, sage1_Wself, sage1_Wneigh, sage1_b, ln_gamma, ln_beta, head_W, head_b):
    raise NotImplementedError("write your pallas kernel here")


# speedup vs baseline: 93.3331x; 93.3331x over previous
import functools

import jax
import jax.numpy as jnp
from jax import lax
from jax.experimental import pallas as pl
from jax.experimental.pallas import tpu as pltpu
from jax.experimental.pallas import tpu_sc as plsc

N = 10000
E = 320000
K = 4
CH = 128
HID = 16
PHI_OUT = 4
PE_DIMS = 8
VOCAB = 200000
OUT = 64
SEED = 1024

NC = 2
NS = 16
NW = NC * NS
EPW = E // NW
BLK = 125
NBLK = EPW // BLK
NPA = N
RPS = NPA // NS

NP_EMB = 10240
EB_BLK = 80
EB_NBLK = NP_EMB // NW // EB_BLK

_mesh = plsc.VectorSubcoreMesh(core_axis_name="c", subcore_axis_name="s")


def _segsum_body(C, with_emb, *refs):
    if with_emb:
        (src_hbm, dst_hbm, h_hbm, ids_hbm, emb_hbm, out_hbm, embout_hbm,
         src_v, dst_v, rows_v, eid_v, erow_v, acc, sem) = refs
    else:
        (src_hbm, dst_hbm, h_hbm, out_hbm,
         src_v, dst_v, rows_v, acc, sem) = refs

    c = lax.axis_index("c")
    s = lax.axis_index("s")
    w = c * NS + s

    pltpu.sync_copy(src_hbm.at[w], src_v)
    pltpu.sync_copy(dst_hbm.at[w], dst_v)

    cq = C // 16

    def zrow(i, carry):
        rows_v[i // cq, pl.ds((i % cq) * 16, 16)] = jnp.zeros((16,), jnp.float32)
        return carry

    lax.fori_loop(0, BLK * cq, zrow, 0)

    def zacc(i, carry):
        pltpu.sync_copy(rows_v, acc.at[pl.ds(s * RPS + i * BLK, BLK)])
        return carry

    lax.fori_loop(0, RPS // BLK, zacc, 0)
    plsc.subcore_barrier()

    def edge(j, carry):
        pltpu.async_copy(h_hbm.at[src_v.at[j]], rows_v, sem).wait()
        pltpu.sync_copy(rows_v, acc.at[dst_v.at[j]], add=True)
        return carry

    lax.fori_loop(0, NBLK, edge, 0)

    if with_emb:
        pltpu.sync_copy(ids_hbm.at[w], eid_v)

        def eblk(t, carry):
            pltpu.async_copy(emb_hbm.at[eid_v.at[t]], erow_v, sem).wait()
            pltpu.sync_copy(
                erow_v, embout_hbm.at[pl.ds(w * (EB_NBLK * EB_BLK) + t * EB_BLK,
                                            EB_BLK)])
            return carry

        lax.fori_loop(0, EB_NBLK, eblk, 0)

    plsc.subcore_barrier()
    pltpu.sync_copy(acc.at[pl.ds(s * RPS, RPS)],
                    out_hbm.at[pl.ds(c * NPA + s * RPS, RPS)])


def _make_segsum(C, with_emb=False):
    out_type = jax.ShapeDtypeStruct((NC * NPA, C), jnp.float32)
    scratch = [
        pltpu.VMEM((NBLK, BLK), jnp.int32),
        pltpu.VMEM((NBLK, BLK), jnp.int32),
        pltpu.VMEM((BLK, C), jnp.float32),
    ]
    if with_emb:
        out_type = (out_type, jax.ShapeDtypeStruct((NP_EMB, CH), jnp.float32))
        scratch += [
            pltpu.VMEM((EB_NBLK, EB_BLK), jnp.int32),
            pltpu.VMEM((EB_BLK, CH), jnp.float32),
        ]
    scratch += [
        pltpu.VMEM_SHARED((NPA, C), jnp.float32),
        pltpu.SemaphoreType.DMA,
    ]
    return pl.kernel(
        functools.partial(_segsum_body, C, with_emb),
        out_type=out_type,
        mesh=_mesh,
        scratch_types=scratch,
        compiler_params=pltpu.CompilerParams(use_tc_tiling_on_sc=False),
    )


_segsum16_emb = _make_segsum(16, with_emb=True)
_segsum128 = _make_segsum(128)


def _mm(a, b):
    return lax.dot(a, b, precision=lax.Precision.HIGHEST)


def _gin0_body(V_ref, P0a_ref, P0b_ref, W1_ref, b1_ref, W2_ref, b2_ref,
               H1_ref):
    A0 = P0a_ref[...] + P0b_ref[...]
    m = V_ref[...] + A0[:, :K]
    for sgn, col in ((1.0, 0), (-1.0, K * HID)):
        t = jax.nn.relu(_mm(sgn * m, W1_ref[...]) + b1_ref[...])
        t = _mm(t, W2_ref[...]) + b2_ref[...]
        H1_ref[:, col:col + K * HID] = jax.nn.relu(t)


NB = 2000
_grid = (N // NB,)
_rows = lambda c: pl.BlockSpec((NB, c), lambda i: (i, 0))
_full = lambda a, b: pl.BlockSpec((a, b), lambda i: (0, 0))

_tc_gin0 = pl.pallas_call(
    _gin0_body,
    grid=_grid,
    in_specs=[_rows(K), _rows(16), _rows(16),
              _full(K, K * HID), _full(1, K * HID),
              _full(K * HID, K * HID), _full(1, K * HID)],
    out_specs=_rows(CH),
    out_shape=jax.ShapeDtypeStruct((N, CH), jnp.float32),
)


def _pe_body(H1_ref, P1a_ref, P1b_ref, x_ref, emb_ref,
             W1_ref, b1_ref, W2_ref, b2_ref,
             rW1_ref, rb1_ref, rW2_ref, rb2_ref,
             peW_ref, peb_ref, h_ref):
    Mf = H1_ref[...] + P1a_ref[...] + P1b_ref[...]
    phi2 = jax.nn.relu(
        _mm(jax.nn.relu(_mm(Mf, W1_ref[...]) + b1_ref[...]), W2_ref[...]) + b2_ref[...])
    phi = phi2[:, :K * PHI_OUT] + phi2[:, K * PHI_OUT:]
    t = jax.nn.relu(_mm(phi, rW1_ref[...]) + rb1_ref[...])
    PE = _mm(t, rW2_ref[...]) + rb2_ref[...]
    h_ref[...] = x_ref[...] + _mm(PE, peW_ref[...]) + peb_ref[...] + emb_ref[...]


_tc_pe = pl.pallas_call(
    _pe_body,
    grid=_grid,
    in_specs=[_rows(CH), _rows(CH), _rows(CH), _rows(CH), _rows(CH),
              _full(CH, CH), _full(1, CH),
              _full(CH, 2 * K * PHI_OUT), _full(1, 2 * K * PHI_OUT),
              _full(K * PHI_OUT, CH), _full(1, CH),
              _full(CH, PE_DIMS), _full(1, PE_DIMS),
              _full(PE_DIMS, CH), _full(1, CH)],
    out_specs=_rows(CH),
    out_shape=jax.ShapeDtypeStruct((N, CH), jnp.float32),
)


def _sage_body(h_ref, Pa_ref, Pb_ref, P0a_ref, P0b_ref,
               Ws_ref, Wn_ref, b_ref, out_ref):
    dinv = 1.0 / jnp.maximum(P0a_ref[:, K:K + 1] + P0b_ref[:, K:K + 1], 1.0)
    neigh = (Pa_ref[...] + Pb_ref[...]) * dinv
    out_ref[...] = jax.nn.relu(
        _mm(h_ref[...], Ws_ref[...]) + _mm(neigh, Wn_ref[...]) + b_ref[...])


_tc_sage = pl.pallas_call(
    _sage_body,
    grid=_grid,
    in_specs=[_rows(CH), _rows(CH), _rows(CH), _rows(16), _rows(16),
              _full(CH, CH), _full(CH, CH), _full(1, CH)],
    out_specs=_rows(CH),
    out_shape=jax.ShapeDtypeStruct((N, CH), jnp.float32),
)


def _head_body(h_ref, P3a_ref, P3b_ref, P0a_ref, P0b_ref,
               Ws_ref, Wn_ref, b_ref, g_ref, beta_ref, hW_ref, hb_ref,
               out_ref):
    dinv = 1.0 / jnp.maximum(P0a_ref[:, K:K + 1] + P0b_ref[:, K:K + 1], 1.0)
    neigh = (P3a_ref[...] + P3b_ref[...]) * dinv
    h2 = jax.nn.relu(
        _mm(h_ref[...], Ws_ref[...]) + _mm(neigh, Wn_ref[...]) + b_ref[...])
    mu = jnp.mean(h2, axis=-1, keepdims=True)
    var = jnp.mean((h2 - mu) * (h2 - mu), axis=-1, keepdims=True)
    hn = (h2 - mu) * lax.rsqrt(var + 1e-5) * g_ref[...] + beta_ref[...]
    out_ref[...] = _mm(hn, hW_ref[...]) + hb_ref[...]


_tc_head = pl.pallas_call(
    _head_body,
    out_shape=jax.ShapeDtypeStruct((SEED, OUT), jnp.float32),
)


def kernel(x, V, edge_index, n_id, emb_table,
           gin0_W1, gin0_b1, gin0_W2, gin0_b2,
           gin1_W1, gin1_b1, gin1_W2, gin1_b2,
           rho_W1, rho_b1, rho_W2, rho_b2,
           pe_W, pe_b,
           sage0_Wself, sage0_Wneigh, sage0_b,
           sage1_Wself, sage1_Wneigh, sage1_b,
           ln_gamma, ln_beta, head_W, head_b):
    f32 = jnp.float32
    src = edge_index[0].reshape(NW, NBLK, BLK)
    dst = edge_index[1].reshape(NW, NBLK, BLK)

    H0 = jnp.concatenate(
        [V, jnp.ones((N, 1), f32), jnp.zeros((N, 16 - K - 1), f32)], axis=1)
    ids = jnp.concatenate(
        [n_id, jnp.zeros((NP_EMB - N,), jnp.int32)]).reshape(NW, EB_NBLK, EB_BLK)

    P0, EMB = _segsum16_emb(src, dst, H0, ids, emb_table)
    P0a, P0b = P0[:N], P0[NPA:NPA + N]

    eyeK = jnp.eye(K, dtype=f32)
    eye2K = jnp.eye(2 * K, dtype=f32)
    row = lambda v: v.reshape(1, -1)

    H1 = _tc_gin0(V, P0a, P0b,
                  jnp.kron(eyeK, gin0_W1), row(jnp.tile(gin0_b1, K)),
                  jnp.kron(eyeK, gin0_W2), row(jnp.tile(gin0_b2, K)))

    P1 = _segsum128(src, dst, H1)

    h = _tc_pe(H1, P1[:N], P1[NPA:NPA + N], x, EMB[:N],
               jnp.kron(eye2K, gin1_W1), row(jnp.tile(gin1_b1, 2 * K)),
               jnp.kron(eye2K, gin1_W2), row(jnp.tile(gin1_b2, 2 * K)),
               jnp.kron(eyeK, rho_W1), row(jnp.tile(rho_b1, K)),
               jnp.concatenate([rho_W2] * K, axis=0), row(K * rho_b2),
               pe_W, row(pe_b))

    P2 = _segsum128(src, dst, h)
    h = _tc_sage(h, P2[:N], P2[NPA:NPA + N], P0a, P0b,
                 sage0_Wself, sage0_Wneigh, row(sage0_b))

    P3 = _segsum128(src, dst, h)
    out = _tc_head(h[:SEED], P3[:SEED], P3[NPA:NPA + SEED],
                   P0a[:SEED], P0b[:SEED],
                   sage1_Wself, sage1_Wneigh, row(sage1_b),
                   row(ln_gamma), row(ln_beta), head_W, row(head_b))
    return out


# speedup vs baseline: 129.7382x; 1.3901x over previous
import functools

import jax
import jax.numpy as jnp
from jax import lax
from jax.experimental import pallas as pl
from jax.experimental.pallas import tpu as pltpu
from jax.experimental.pallas import tpu_sc as plsc

N = 10000
E = 320000
K = 4
CH = 128
HID = 16
PHI_OUT = 4
PE_DIMS = 8
VOCAB = 200000
OUT = 64
SEED = 1024

NC = 2
NS = 16
NW = NC * NS
EPW = E // NW
BLK = 100
NBLK = EPW // BLK
NPA = N
RPS = NPA // NS

NP_EMB = 10240
EB_BLK = 80
EB_NBLK = NP_EMB // NW // EB_BLK

_mesh = plsc.VectorSubcoreMesh(core_axis_name="c", subcore_axis_name="s")


def _segsum_body(C, with_emb, *refs):
    if with_emb:
        (src_hbm, dst_hbm, h_hbm, ids_hbm, emb_hbm, out_hbm, embout_hbm,
         src_v, dst_v, rows0_v, rows1_v, eid_v, erow_v, acc,
         sem0, sem1) = refs
    else:
        (src_hbm, dst_hbm, h_hbm, out_hbm,
         src_v, dst_v, rows0_v, rows1_v, acc, sem0, sem1) = refs

    c = lax.axis_index("c")
    s = lax.axis_index("s")
    w = c * NS + s

    pltpu.sync_copy(src_hbm.at[w], src_v)
    pltpu.sync_copy(dst_hbm.at[w], dst_v)

    cq = C // 16

    def zrow(i, carry):
        rows0_v[i // cq, pl.ds((i % cq) * 16, 16)] = jnp.zeros((16,), jnp.float32)
        return carry

    lax.fori_loop(0, BLK * cq, zrow, 0)

    def zacc(i, carry):
        pltpu.sync_copy(rows0_v, acc.at[pl.ds(s * RPS + i * BLK, BLK)])
        return carry

    lax.fori_loop(0, RPS // BLK, zacc, 0)
    pltpu.sync_copy(rows0_v.at[pl.ds(0, RPS % BLK)],
                    acc.at[pl.ds(s * RPS + (RPS // BLK) * BLK, RPS % BLK)])
    plsc.subcore_barrier()

    def gather(j, buf, sem):
        pltpu.async_copy(h_hbm.at[src_v.at[j]], buf, sem)

    def gwait(buf, sem):
        pltpu.make_async_copy(h_hbm.at[src_v.at[0]], buf, sem).wait()

    def scat(j, buf):
        pltpu.sync_copy(buf, acc.at[dst_v.at[j]], add=True)

    gather(0, rows0_v, sem0)
    gather(1, rows1_v, sem1)

    def edge(i, carry):
        j = i * 2
        gwait(rows0_v, sem0)
        scat(j, rows0_v)
        gather(j + 2, rows0_v, sem0)
        gwait(rows1_v, sem1)
        scat(j + 1, rows1_v)
        gather(j + 3, rows1_v, sem1)
        return carry

    lax.fori_loop(0, NBLK // 2 - 1, edge, 0)
    gwait(rows0_v, sem0)
    scat(NBLK - 2, rows0_v)
    gwait(rows1_v, sem1)
    scat(NBLK - 1, rows1_v)

    if with_emb:
        pltpu.sync_copy(ids_hbm.at[w], eid_v)

        def eblk(t, carry):
            pltpu.async_copy(emb_hbm.at[eid_v.at[t]], erow_v, sem0).wait()
            pltpu.sync_copy(
                erow_v, embout_hbm.at[pl.ds(w * (EB_NBLK * EB_BLK) + t * EB_BLK,
                                            EB_BLK)])
            return carry

        lax.fori_loop(0, EB_NBLK, eblk, 0)

    plsc.subcore_barrier()
    pltpu.sync_copy(acc.at[pl.ds(s * RPS, RPS)],
                    out_hbm.at[pl.ds(c * NPA + s * RPS, RPS)])


def _make_segsum(C, with_emb=False):
    out_type = jax.ShapeDtypeStruct((NC * NPA, C), jnp.float32)
    scratch = [
        pltpu.VMEM((NBLK, BLK), jnp.int32),
        pltpu.VMEM((NBLK, BLK), jnp.int32),
        pltpu.VMEM((BLK, C), jnp.float32),
        pltpu.VMEM((BLK, C), jnp.float32),
    ]
    if with_emb:
        out_type = (out_type, jax.ShapeDtypeStruct((NP_EMB, CH), jnp.float32))
        scratch += [
            pltpu.VMEM((EB_NBLK, EB_BLK), jnp.int32),
            pltpu.VMEM((EB_BLK, CH), jnp.float32),
        ]
    scratch += [
        pltpu.VMEM_SHARED((NPA, C), jnp.float32),
        pltpu.SemaphoreType.DMA,
        pltpu.SemaphoreType.DMA,
    ]
    return pl.kernel(
        functools.partial(_segsum_body, C, with_emb),
        out_type=out_type,
        mesh=_mesh,
        scratch_types=scratch,
        compiler_params=pltpu.CompilerParams(use_tc_tiling_on_sc=False),
    )


_segsum16_emb = _make_segsum(16, with_emb=True)
_segsum128 = _make_segsum(128)


def _mm(a, b):
    return lax.dot(a, b, precision=lax.Precision.HIGHEST)


def _gin0_body(V_ref, P0a_ref, P0b_ref, W1_ref, b1_ref, W2_ref, b2_ref,
               H1_ref):
    A0 = P0a_ref[...] + P0b_ref[...]
    m = V_ref[...] + A0[:, :K]
    for sgn, col in ((1.0, 0), (-1.0, K * HID)):
        t = jax.nn.relu(_mm(sgn * m, W1_ref[...]) + b1_ref[...])
        t = _mm(t, W2_ref[...]) + b2_ref[...]
        H1_ref[:, col:col + K * HID] = jax.nn.relu(t)


NB = 2000
_grid = (N // NB,)
_rows = lambda c: pl.BlockSpec((NB, c), lambda i: (i, 0))
_full = lambda a, b: pl.BlockSpec((a, b), lambda i: (0, 0))

_tc_gin0 = pl.pallas_call(
    _gin0_body,
    grid=_grid,
    in_specs=[_rows(K), _rows(16), _rows(16),
              _full(K, K * HID), _full(1, K * HID),
              _full(K * HID, K * HID), _full(1, K * HID)],
    out_specs=_rows(CH),
    out_shape=jax.ShapeDtypeStruct((N, CH), jnp.float32),
)


def _pe_body(H1_ref, P1a_ref, P1b_ref, x_ref, emb_ref,
             W1_ref, b1_ref, W2_ref, b2_ref,
             rW1_ref, rb1_ref, rW2_ref, rb2_ref,
             peW_ref, peb_ref, h_ref):
    Mf = H1_ref[...] + P1a_ref[...] + P1b_ref[...]
    phi2 = jax.nn.relu(
        _mm(jax.nn.relu(_mm(Mf, W1_ref[...]) + b1_ref[...]), W2_ref[...]) + b2_ref[...])
    phi = phi2[:, :K * PHI_OUT] + phi2[:, K * PHI_OUT:]
    t = jax.nn.relu(_mm(phi, rW1_ref[...]) + rb1_ref[...])
    PE = _mm(t, rW2_ref[...]) + rb2_ref[...]
    h_ref[...] = x_ref[...] + _mm(PE, peW_ref[...]) + peb_ref[...] + emb_ref[...]


_tc_pe = pl.pallas_call(
    _pe_body,
    grid=_grid,
    in_specs=[_rows(CH), _rows(CH), _rows(CH), _rows(CH), _rows(CH),
              _full(CH, CH), _full(1, CH),
              _full(CH, 2 * K * PHI_OUT), _full(1, 2 * K * PHI_OUT),
              _full(K * PHI_OUT, CH), _full(1, CH),
              _full(CH, PE_DIMS), _full(1, PE_DIMS),
              _full(PE_DIMS, CH), _full(1, CH)],
    out_specs=_rows(CH),
    out_shape=jax.ShapeDtypeStruct((N, CH), jnp.float32),
)


def _sage_body(h_ref, Pa_ref, Pb_ref, P0a_ref, P0b_ref,
               Ws_ref, Wn_ref, b_ref, out_ref):
    dinv = 1.0 / jnp.maximum(P0a_ref[:, K:K + 1] + P0b_ref[:, K:K + 1], 1.0)
    neigh = (Pa_ref[...] + Pb_ref[...]) * dinv
    out_ref[...] = jax.nn.relu(
        _mm(h_ref[...], Ws_ref[...]) + _mm(neigh, Wn_ref[...]) + b_ref[...])


_tc_sage = pl.pallas_call(
    _sage_body,
    grid=_grid,
    in_specs=[_rows(CH), _rows(CH), _rows(CH), _rows(16), _rows(16),
              _full(CH, CH), _full(CH, CH), _full(1, CH)],
    out_specs=_rows(CH),
    out_shape=jax.ShapeDtypeStruct((N, CH), jnp.float32),
)


def _head_body(h_ref, P3a_ref, P3b_ref, P0a_ref, P0b_ref,
               Ws_ref, Wn_ref, b_ref, g_ref, beta_ref, hW_ref, hb_ref,
               out_ref):
    dinv = 1.0 / jnp.maximum(P0a_ref[:, K:K + 1] + P0b_ref[:, K:K + 1], 1.0)
    neigh = (P3a_ref[...] + P3b_ref[...]) * dinv
    h2 = jax.nn.relu(
        _mm(h_ref[...], Ws_ref[...]) + _mm(neigh, Wn_ref[...]) + b_ref[...])
    mu = jnp.mean(h2, axis=-1, keepdims=True)
    var = jnp.mean((h2 - mu) * (h2 - mu), axis=-1, keepdims=True)
    hn = (h2 - mu) * lax.rsqrt(var + 1e-5) * g_ref[...] + beta_ref[...]
    out_ref[...] = _mm(hn, hW_ref[...]) + hb_ref[...]


_tc_head = pl.pallas_call(
    _head_body,
    out_shape=jax.ShapeDtypeStruct((SEED, OUT), jnp.float32),
)


def kernel(x, V, edge_index, n_id, emb_table,
           gin0_W1, gin0_b1, gin0_W2, gin0_b2,
           gin1_W1, gin1_b1, gin1_W2, gin1_b2,
           rho_W1, rho_b1, rho_W2, rho_b2,
           pe_W, pe_b,
           sage0_Wself, sage0_Wneigh, sage0_b,
           sage1_Wself, sage1_Wneigh, sage1_b,
           ln_gamma, ln_beta, head_W, head_b):
    f32 = jnp.float32
    src = edge_index[0].reshape(NW, NBLK, BLK)
    dst = edge_index[1].reshape(NW, NBLK, BLK)

    H0 = jnp.concatenate(
        [V, jnp.ones((N, 1), f32), jnp.zeros((N, 16 - K - 1), f32)], axis=1)
    ids = jnp.concatenate(
        [n_id, jnp.zeros((NP_EMB - N,), jnp.int32)]).reshape(NW, EB_NBLK, EB_BLK)

    P0, EMB = _segsum16_emb(src, dst, H0, ids, emb_table)
    P0a, P0b = P0[:N], P0[NPA:NPA + N]

    eyeK = jnp.eye(K, dtype=f32)
    eye2K = jnp.eye(2 * K, dtype=f32)
    row = lambda v: v.reshape(1, -1)

    H1 = _tc_gin0(V, P0a, P0b,
                  jnp.kron(eyeK, gin0_W1), row(jnp.tile(gin0_b1, K)),
                  jnp.kron(eyeK, gin0_W2), row(jnp.tile(gin0_b2, K)))

    P1 = _segsum128(src, dst, H1)

    h = _tc_pe(H1, P1[:N], P1[NPA:NPA + N], x, EMB[:N],
               jnp.kron(eye2K, gin1_W1), row(jnp.tile(gin1_b1, 2 * K)),
               jnp.kron(eye2K, gin1_W2), row(jnp.tile(gin1_b2, 2 * K)),
               jnp.kron(eyeK, rho_W1), row(jnp.tile(rho_b1, K)),
               jnp.concatenate([rho_W2] * K, axis=0), row(K * rho_b2),
               pe_W, row(pe_b))

    P2 = _segsum128(src, dst, h)
    h = _tc_sage(h, P2[:N], P2[NPA:NPA + N], P0a, P0b,
                 sage0_Wself, sage0_Wneigh, row(sage0_b))

    P3 = _segsum128(src, dst, h)
    out = _tc_head(h[:SEED], P3[:SEED], P3[NPA:NPA + SEED],
                   P0a[:SEED], P0b[:SEED],
                   sage1_Wself, sage1_Wneigh, row(sage1_b),
                   row(ln_gamma), row(ln_beta), head_W, row(head_b))
    return out


# speedup vs baseline: 133.0098x; 1.0252x over previous
import functools

import jax
import jax.numpy as jnp
from jax import lax
from jax.experimental import pallas as pl
from jax.experimental.pallas import tpu as pltpu
from jax.experimental.pallas import tpu_sc as plsc

N = 10000
E = 320000
K = 4
CH = 128
HID = 16
PHI_OUT = 4
PE_DIMS = 8
VOCAB = 200000
OUT = 64
SEED = 1024

NC = 2
NS = 16
NW = NC * NS
EPW = E // NW
BLK = 100
NBLK = EPW // BLK
NPA = N
RPS = NPA // NS

NP_EMB = 10240
EB_BLK = 80
EB_NBLK = NP_EMB // NW // EB_BLK

_mesh = plsc.VectorSubcoreMesh(core_axis_name="c", subcore_axis_name="s")


def _segsum_body(C, with_emb, *refs):
    if with_emb:
        (src_hbm, dst_hbm, h_hbm, ids_hbm, emb_hbm, out_hbm, embout_hbm,
         src_v, dst_v, rows0_v, rows1_v, eid_v, erow_v, acc,
         sem0, sem1) = refs
    else:
        (src_hbm, dst_hbm, h_hbm, out_hbm,
         src_v, dst_v, rows0_v, rows1_v, acc, sem0, sem1) = refs

    c = lax.axis_index("c")
    s = lax.axis_index("s")
    w = c * NS + s

    pltpu.sync_copy(src_hbm.at[w], src_v)
    pltpu.sync_copy(dst_hbm.at[w], dst_v)

    cq = C // 16

    def zrow(i, carry):
        rows0_v[i // cq, pl.ds((i % cq) * 16, 16)] = jnp.zeros((16,), jnp.float32)
        return carry

    lax.fori_loop(0, BLK * cq, zrow, 0)

    def zacc(i, carry):
        pltpu.sync_copy(rows0_v, acc.at[pl.ds(s * RPS + i * BLK, BLK)])
        return carry

    lax.fori_loop(0, RPS // BLK, zacc, 0)
    pltpu.sync_copy(rows0_v.at[pl.ds(0, RPS % BLK)],
                    acc.at[pl.ds(s * RPS + (RPS // BLK) * BLK, RPS % BLK)])
    plsc.subcore_barrier()

    def gather(j, buf, sem):
        pltpu.async_copy(h_hbm.at[src_v.at[j]], buf, sem)

    def gwait(buf, sem):
        pltpu.make_async_copy(h_hbm.at[src_v.at[0]], buf, sem).wait()

    def scat(j, buf):
        pltpu.sync_copy(buf, acc.at[dst_v.at[j]], add=True)

    gather(0, rows0_v, sem0)
    gather(1, rows1_v, sem1)

    def edge(i, carry):
        j = i * 2
        gwait(rows0_v, sem0)
        scat(j, rows0_v)
        gather(j + 2, rows0_v, sem0)
        gwait(rows1_v, sem1)
        scat(j + 1, rows1_v)
        gather(j + 3, rows1_v, sem1)
        return carry

    lax.fori_loop(0, NBLK // 2 - 1, edge, 0)
    gwait(rows0_v, sem0)
    scat(NBLK - 2, rows0_v)
    gwait(rows1_v, sem1)
    scat(NBLK - 1, rows1_v)

    if with_emb:
        pltpu.sync_copy(ids_hbm.at[w], eid_v)

        def eblk(t, carry):
            pltpu.async_copy(emb_hbm.at[eid_v.at[t]], erow_v, sem0).wait()
            pltpu.sync_copy(
                erow_v, embout_hbm.at[pl.ds(w * (EB_NBLK * EB_BLK) + t * EB_BLK,
                                            EB_BLK)])
            return carry

        lax.fori_loop(0, EB_NBLK, eblk, 0)

    plsc.subcore_barrier()
    pltpu.sync_copy(acc.at[pl.ds(s * RPS, RPS)],
                    out_hbm.at[pl.ds(c * NPA + s * RPS, RPS)])


NP3 = 1040
RP3 = NP3 // NS
SB = 128
SNB = 79


def _seed_body(src_hbm, dst_hbm, h_hbm, out_hbm,
               srcf_v, dstf_v, csrc_v, cdst_v, rows0_v, rows1_v, acc,
               sem0, sem1):
    c = lax.axis_index("c")
    s = lax.axis_index("s")
    w = c * NS + s

    pltpu.sync_copy(src_hbm.at[w], srcf_v)
    pltpu.sync_copy(dst_hbm.at[w], dstf_v)

    zero16 = jnp.zeros((16,), jnp.int32)
    dummy16 = jnp.full((16,), SEED, jnp.int32)

    def pf(i, carry):
        csrc_v[i // 8, pl.ds((i % 8) * 16, 16)] = zero16
        cdst_v[i // 8, pl.ds((i % 8) * 16, 16)] = dummy16
        return carry

    lax.fori_loop(0, SNB * 8, pf, 0)

    def comp(i, base):
        d16 = dstf_v[pl.ds(i * 16, 16)]
        s16 = srcf_v[pl.ds(i * 16, 16)]
        m = d16 < SEED
        mi = jnp.where(m, 1, 0).astype(jnp.int32)
        pos = base + plsc.cumsum(mi) - mi
        pdiv = lax.shift_right_logical(pos, 7)
        pmod = lax.bitwise_and(pos, 127)
        plsc.store_scatter(csrc_v, [pdiv, pmod], s16, mask=m)
        plsc.store_scatter(cdst_v, [pdiv, pmod], d16, mask=m)
        return base + plsc.all_reduce_population_count(m)

    base = lax.fori_loop(0, EPW // 16, comp, jnp.zeros((16,), jnp.int32))
    cnt = lax.reduce_max(base, axes=(0,))
    nb = (cnt + (SB - 1)) // SB

    def zrow(i, carry):
        rows0_v[i // 8, pl.ds((i % 8) * 16, 16)] = jnp.zeros((16,), jnp.float32)
        return carry

    lax.fori_loop(0, RP3 * 8, zrow, 0)
    pltpu.sync_copy(rows0_v.at[pl.ds(0, RP3)], acc.at[pl.ds(s * RP3, RP3)])
    plsc.subcore_barrier()

    def gwait(buf, sem):
        pltpu.make_async_copy(h_hbm.at[csrc_v.at[0]], buf, sem).wait()

    @pl.when(nb > 0)
    def _():
        pltpu.async_copy(h_hbm.at[csrc_v.at[0]], rows0_v, sem0)

    def blk(j, carry):
        even = lax.rem(j, 2) == 0

        @pl.when(even)
        def _():
            gwait(rows0_v, sem0)

            @pl.when(j + 1 < nb)
            def _():
                pltpu.async_copy(h_hbm.at[csrc_v.at[j + 1]], rows1_v, sem1)

            pltpu.sync_copy(rows0_v, acc.at[cdst_v.at[j]], add=True)

        @pl.when(jnp.logical_not(even))
        def _():
            gwait(rows1_v, sem1)

            @pl.when(j + 1 < nb)
            def _():
                pltpu.async_copy(h_hbm.at[csrc_v.at[j + 1]], rows0_v, sem0)

            pltpu.sync_copy(rows1_v, acc.at[cdst_v.at[j]], add=True)

        return carry

    lax.fori_loop(0, nb, blk, 0)

    plsc.subcore_barrier()
    pltpu.sync_copy(acc.at[pl.ds(s * RP3, RP3)],
                    out_hbm.at[pl.ds(c * NP3 + s * RP3, RP3)])


_segsum_seed = pl.kernel(
    _seed_body,
    out_type=jax.ShapeDtypeStruct((NC * NP3, CH), jnp.float32),
    mesh=_mesh,
    scratch_types=[
        pltpu.VMEM((EPW,), jnp.int32),
        pltpu.VMEM((EPW,), jnp.int32),
        pltpu.VMEM((SNB, SB), jnp.int32),
        pltpu.VMEM((SNB, SB), jnp.int32),
        pltpu.VMEM((SB, CH), jnp.float32),
        pltpu.VMEM((SB, CH), jnp.float32),
        pltpu.VMEM_SHARED((NP3, CH), jnp.float32),
        pltpu.SemaphoreType.DMA,
        pltpu.SemaphoreType.DMA,
    ],
    compiler_params=pltpu.CompilerParams(use_tc_tiling_on_sc=False,
                                         needs_layout_passes=False),
)


def _make_segsum(C, with_emb=False):
    out_type = jax.ShapeDtypeStruct((NC * NPA, C), jnp.float32)
    scratch = [
        pltpu.VMEM((NBLK, BLK), jnp.int32),
        pltpu.VMEM((NBLK, BLK), jnp.int32),
        pltpu.VMEM((BLK, C), jnp.float32),
        pltpu.VMEM((BLK, C), jnp.float32),
    ]
    if with_emb:
        out_type = (out_type, jax.ShapeDtypeStruct((NP_EMB, CH), jnp.float32))
        scratch += [
            pltpu.VMEM((EB_NBLK, EB_BLK), jnp.int32),
            pltpu.VMEM((EB_BLK, CH), jnp.float32),
        ]
    scratch += [
        pltpu.VMEM_SHARED((NPA, C), jnp.float32),
        pltpu.SemaphoreType.DMA,
        pltpu.SemaphoreType.DMA,
    ]
    return pl.kernel(
        functools.partial(_segsum_body, C, with_emb),
        out_type=out_type,
        mesh=_mesh,
        scratch_types=scratch,
        compiler_params=pltpu.CompilerParams(use_tc_tiling_on_sc=False),
    )


_segsum16_emb = _make_segsum(16, with_emb=True)
_segsum128 = _make_segsum(128)


def _mm(a, b):
    return lax.dot(a, b, precision=lax.Precision.HIGHEST)


def _gin0_body(V_ref, P0a_ref, P0b_ref, W1_ref, b1_ref, W2_ref, b2_ref,
               H1_ref):
    A0 = P0a_ref[...] + P0b_ref[...]
    m = V_ref[...] + A0[:, :K]
    for sgn, col in ((1.0, 0), (-1.0, K * HID)):
        t = jax.nn.relu(_mm(sgn * m, W1_ref[...]) + b1_ref[...])
        t = _mm(t, W2_ref[...]) + b2_ref[...]
        H1_ref[:, col:col + K * HID] = jax.nn.relu(t)


NB = 2000
_grid = (N // NB,)
_rows = lambda c: pl.BlockSpec((NB, c), lambda i: (i, 0))
_full = lambda a, b: pl.BlockSpec((a, b), lambda i: (0, 0))

_tc_gin0 = pl.pallas_call(
    _gin0_body,
    grid=_grid,
    in_specs=[_rows(K), _rows(16), _rows(16),
              _full(K, K * HID), _full(1, K * HID),
              _full(K * HID, K * HID), _full(1, K * HID)],
    out_specs=_rows(CH),
    out_shape=jax.ShapeDtypeStruct((N, CH), jnp.float32),
)


def _pe_body(H1_ref, P1a_ref, P1b_ref, x_ref, emb_ref,
             W1_ref, b1_ref, W2_ref, b2_ref,
             rW1_ref, rb1_ref, rW2_ref, rb2_ref,
             peW_ref, peb_ref, h_ref):
    Mf = H1_ref[...] + P1a_ref[...] + P1b_ref[...]
    phi2 = jax.nn.relu(
        _mm(jax.nn.relu(_mm(Mf, W1_ref[...]) + b1_ref[...]), W2_ref[...]) + b2_ref[...])
    phi = phi2[:, :K * PHI_OUT] + phi2[:, K * PHI_OUT:]
    t = jax.nn.relu(_mm(phi, rW1_ref[...]) + rb1_ref[...])
    PE = _mm(t, rW2_ref[...]) + rb2_ref[...]
    h_ref[...] = x_ref[...] + _mm(PE, peW_ref[...]) + peb_ref[...] + emb_ref[...]


_tc_pe = pl.pallas_call(
    _pe_body,
    grid=_grid,
    in_specs=[_rows(CH), _rows(CH), _rows(CH), _rows(CH), _rows(CH),
              _full(CH, CH), _full(1, CH),
              _full(CH, 2 * K * PHI_OUT), _full(1, 2 * K * PHI_OUT),
              _full(K * PHI_OUT, CH), _full(1, CH),
              _full(CH, PE_DIMS), _full(1, PE_DIMS),
              _full(PE_DIMS, CH), _full(1, CH)],
    out_specs=_rows(CH),
    out_shape=jax.ShapeDtypeStruct((N, CH), jnp.float32),
)


def _sage_body(h_ref, Pa_ref, Pb_ref, P0a_ref, P0b_ref,
               Ws_ref, Wn_ref, b_ref, out_ref):
    dinv = 1.0 / jnp.maximum(P0a_ref[:, K:K + 1] + P0b_ref[:, K:K + 1], 1.0)
    neigh = (Pa_ref[...] + Pb_ref[...]) * dinv
    out_ref[...] = jax.nn.relu(
        _mm(h_ref[...], Ws_ref[...]) + _mm(neigh, Wn_ref[...]) + b_ref[...])


_tc_sage = pl.pallas_call(
    _sage_body,
    grid=_grid,
    in_specs=[_rows(CH), _rows(CH), _rows(CH), _rows(16), _rows(16),
              _full(CH, CH), _full(CH, CH), _full(1, CH)],
    out_specs=_rows(CH),
    out_shape=jax.ShapeDtypeStruct((N, CH), jnp.float32),
)


def _head_body(h_ref, P3a_ref, P3b_ref, P0a_ref, P0b_ref,
               Ws_ref, Wn_ref, b_ref, g_ref, beta_ref, hW_ref, hb_ref,
               out_ref):
    dinv = 1.0 / jnp.maximum(P0a_ref[:, K:K + 1] + P0b_ref[:, K:K + 1], 1.0)
    neigh = (P3a_ref[...] + P3b_ref[...]) * dinv
    h2 = jax.nn.relu(
        _mm(h_ref[...], Ws_ref[...]) + _mm(neigh, Wn_ref[...]) + b_ref[...])
    mu = jnp.mean(h2, axis=-1, keepdims=True)
    var = jnp.mean((h2 - mu) * (h2 - mu), axis=-1, keepdims=True)
    hn = (h2 - mu) * lax.rsqrt(var + 1e-5) * g_ref[...] + beta_ref[...]
    out_ref[...] = _mm(hn, hW_ref[...]) + hb_ref[...]


_tc_head = pl.pallas_call(
    _head_body,
    out_shape=jax.ShapeDtypeStruct((SEED, OUT), jnp.float32),
)


def kernel(x, V, edge_index, n_id, emb_table,
           gin0_W1, gin0_b1, gin0_W2, gin0_b2,
           gin1_W1, gin1_b1, gin1_W2, gin1_b2,
           rho_W1, rho_b1, rho_W2, rho_b2,
           pe_W, pe_b,
           sage0_Wself, sage0_Wneigh, sage0_b,
           sage1_Wself, sage1_Wneigh, sage1_b,
           ln_gamma, ln_beta, head_W, head_b):
    f32 = jnp.float32
    src = edge_index[0].reshape(NW, NBLK, BLK)
    dst = edge_index[1].reshape(NW, NBLK, BLK)

    H0 = jnp.concatenate(
        [V, jnp.ones((N, 1), f32), jnp.zeros((N, 16 - K - 1), f32)], axis=1)
    ids = jnp.concatenate(
        [n_id, jnp.zeros((NP_EMB - N,), jnp.int32)]).reshape(NW, EB_NBLK, EB_BLK)

    P0, EMB = _segsum16_emb(src, dst, H0, ids, emb_table)
    P0a, P0b = P0[:N], P0[NPA:NPA + N]

    eyeK = jnp.eye(K, dtype=f32)
    eye2K = jnp.eye(2 * K, dtype=f32)
    row = lambda v: v.reshape(1, -1)

    H1 = _tc_gin0(V, P0a, P0b,
                  jnp.kron(eyeK, gin0_W1), row(jnp.tile(gin0_b1, K)),
                  jnp.kron(eyeK, gin0_W2), row(jnp.tile(gin0_b2, K)))

    P1 = _segsum128(src, dst, H1)

    h = _tc_pe(H1, P1[:N], P1[NPA:NPA + N], x, EMB[:N],
               jnp.kron(eye2K, gin1_W1), row(jnp.tile(gin1_b1, 2 * K)),
               jnp.kron(eye2K, gin1_W2), row(jnp.tile(gin1_b2, 2 * K)),
               jnp.kron(eyeK, rho_W1), row(jnp.tile(rho_b1, K)),
               jnp.concatenate([rho_W2] * K, axis=0), row(K * rho_b2),
               pe_W, row(pe_b))

    P2 = _segsum128(src, dst, h)
    h = _tc_sage(h, P2[:N], P2[NPA:NPA + N], P0a, P0b,
                 sage0_Wself, sage0_Wneigh, row(sage0_b))

    P3 = _segsum_seed(edge_index[0].reshape(NW, EPW),
                      edge_index[1].reshape(NW, EPW), h)
    out = _tc_head(h[:SEED], P3[:SEED], P3[NP3:NP3 + SEED],
                   P0a[:SEED], P0b[:SEED],
                   sage1_Wself, sage1_Wneigh, row(sage1_b),
                   row(ln_gamma), row(ln_beta), head_W, row(head_b))
    return out


# speedup vs baseline: 136.1861x; 1.0239x over previous
import functools

import jax
import jax.numpy as jnp
from jax import lax
from jax.experimental import pallas as pl
from jax.experimental.pallas import tpu as pltpu
from jax.experimental.pallas import tpu_sc as plsc

N = 10000
E = 320000
K = 4
CH = 128
HID = 16
PHI_OUT = 4
PE_DIMS = 8
VOCAB = 200000
OUT = 64
SEED = 1024

NC = 2
NS = 16
NW = NC * NS
EPW = E // NW
BLK = 100
NBLK = EPW // BLK
NPA = N
RPS = NPA // NS

NP_EMB = 10240
EB_BLK = 80
EB_NBLK = NP_EMB // NW // EB_BLK

_mesh = plsc.VectorSubcoreMesh(core_axis_name="c", subcore_axis_name="s")


def _segsum_body(C, with_emb, *refs):
    if with_emb:
        (src_hbm, dst_hbm, h_hbm, ids_hbm, emb_hbm, out_hbm, embout_hbm,
         src_v, dst_v, rows0_v, rows1_v, rows2_v, rows3_v, eid_v, erow_v, acc,
         sem0, sem1, sem2, sem3, ss0, ss1, ss2, ss3) = refs
        bufs = (rows0_v, rows1_v, rows2_v, rows3_v)
        gsems = (sem0, sem1, sem2, sem3)
        ssems = (ss0, ss1, ss2, ss3)
    else:
        (src_hbm, dst_hbm, h_hbm, out_hbm,
         src_v, dst_v, rows0_v, rows1_v, acc, sem0, sem1) = refs

    c = lax.axis_index("c")
    s = lax.axis_index("s")
    w = c * NS + s

    pltpu.sync_copy(src_hbm.at[w], src_v)
    pltpu.sync_copy(dst_hbm.at[w], dst_v)

    cq = C // 16

    def zrow(i, carry):
        rows0_v[i // cq, pl.ds((i % cq) * 16, 16)] = jnp.zeros((16,), jnp.float32)
        return carry

    lax.fori_loop(0, BLK * cq, zrow, 0)

    def zacc(i, carry):
        pltpu.sync_copy(rows0_v, acc.at[pl.ds(s * RPS + i * BLK, BLK)])
        return carry

    lax.fori_loop(0, RPS // BLK, zacc, 0)
    pltpu.sync_copy(rows0_v.at[pl.ds(0, RPS % BLK)],
                    acc.at[pl.ds(s * RPS + (RPS // BLK) * BLK, RPS % BLK)])
    plsc.subcore_barrier()

    def gather(j, buf, sem):
        pltpu.async_copy(h_hbm.at[src_v.at[j]], buf, sem)

    def gwait(buf, sem):
        pltpu.make_async_copy(h_hbm.at[src_v.at[0]], buf, sem).wait()

    def scat(j, buf):
        pltpu.sync_copy(buf, acc.at[dst_v.at[j]], add=True)

    if with_emb:
        def ascat(j, buf, sem):
            pltpu.async_copy(buf, acc.at[dst_v.at[j]], sem, add=True)

        def sdrain(buf, sem):
            pltpu.make_async_copy(buf, acc.at[dst_v.at[0]], sem).wait()

        gather(0, bufs[0], gsems[0])
        gather(1, bufs[1], gsems[1])

        def edge4(o, carry):
            for b in range(4):
                j = o * 4 + b
                bb = (b + 2) % 4
                gwait(bufs[b], gsems[b])
                ascat(j, bufs[b], ssems[b])

                @pl.when(j + 2 < NBLK)
                def _():
                    @pl.when(j >= 2)
                    def _():
                        sdrain(bufs[bb], ssems[bb])
                    gather(j + 2, bufs[bb], gsems[bb])
            return carry

        lax.fori_loop(0, NBLK // 4, edge4, 0)
        for b in range(4):
            sdrain(bufs[b], ssems[b])
    else:
        gather(0, rows0_v, sem0)
        gather(1, rows1_v, sem1)

        def edge(i, carry):
            j = i * 2
            gwait(rows0_v, sem0)
            scat(j, rows0_v)
            gather(j + 2, rows0_v, sem0)
            gwait(rows1_v, sem1)
            scat(j + 1, rows1_v)
            gather(j + 3, rows1_v, sem1)
            return carry

        lax.fori_loop(0, NBLK // 2 - 1, edge, 0)
        gwait(rows0_v, sem0)
        scat(NBLK - 2, rows0_v)
        gwait(rows1_v, sem1)
        scat(NBLK - 1, rows1_v)

    if with_emb:
        pltpu.sync_copy(ids_hbm.at[w], eid_v)

        def eblk(t, carry):
            pltpu.async_copy(emb_hbm.at[eid_v.at[t]], erow_v, sem0).wait()
            pltpu.sync_copy(
                erow_v, embout_hbm.at[pl.ds(w * (EB_NBLK * EB_BLK) + t * EB_BLK,
                                            EB_BLK)])
            return carry

        lax.fori_loop(0, EB_NBLK, eblk, 0)

    plsc.subcore_barrier()
    pltpu.sync_copy(acc.at[pl.ds(s * RPS, RPS)],
                    out_hbm.at[pl.ds(c * NPA + s * RPS, RPS)])


NP3 = 1040
RP3 = NP3 // NS
SB = 128
SNB = 79


def _seed_body(src_hbm, dst_hbm, h_hbm, out_hbm,
               srcf_v, dstf_v, csrc_v, cdst_v, rows0_v, rows1_v, acc,
               sem0, sem1):
    c = lax.axis_index("c")
    s = lax.axis_index("s")
    w = c * NS + s

    pltpu.sync_copy(src_hbm.at[w], srcf_v)
    pltpu.sync_copy(dst_hbm.at[w], dstf_v)

    zero16 = jnp.zeros((16,), jnp.int32)
    dummy16 = jnp.full((16,), SEED, jnp.int32)

    @plsc.parallel_loop(0, SNB * 8, unroll=4)
    def _(i):
        csrc_v[i // 8, pl.ds((i % 8) * 16, 16)] = zero16
        cdst_v[i // 8, pl.ds((i % 8) * 16, 16)] = dummy16

    @plsc.parallel_loop(0, EPW // 16, unroll=4,
                        carry=jnp.zeros((16,), jnp.int32))
    def base(i, b):
        d16 = dstf_v[pl.ds(i * 16, 16)]
        s16 = srcf_v[pl.ds(i * 16, 16)]
        m = d16 < SEED
        mi = jnp.where(m, 1, 0).astype(jnp.int32)
        pos = b + plsc.cumsum(mi) - mi
        pdiv = lax.shift_right_logical(pos, 7)
        pmod = lax.bitwise_and(pos, 127)
        plsc.store_scatter(csrc_v, [pdiv, pmod], s16, mask=m)
        plsc.store_scatter(cdst_v, [pdiv, pmod], d16, mask=m)
        return b + plsc.all_reduce_population_count(m)
    cnt = lax.reduce_max(base, axes=(0,))
    nb = (cnt + (SB - 1)) // SB

    def zrow(i, carry):
        rows0_v[i // 8, pl.ds((i % 8) * 16, 16)] = jnp.zeros((16,), jnp.float32)
        return carry

    lax.fori_loop(0, RP3 * 8, zrow, 0)
    pltpu.sync_copy(rows0_v.at[pl.ds(0, RP3)], acc.at[pl.ds(s * RP3, RP3)])
    plsc.subcore_barrier()

    def gwait(buf, sem):
        pltpu.make_async_copy(h_hbm.at[csrc_v.at[0]], buf, sem).wait()

    @pl.when(nb > 0)
    def _():
        pltpu.async_copy(h_hbm.at[csrc_v.at[0]], rows0_v, sem0)

    def blk(j, carry):
        even = lax.rem(j, 2) == 0

        @pl.when(even)
        def _():
            gwait(rows0_v, sem0)

            @pl.when(j + 1 < nb)
            def _():
                pltpu.async_copy(h_hbm.at[csrc_v.at[j + 1]], rows1_v, sem1)

            pltpu.sync_copy(rows0_v, acc.at[cdst_v.at[j]], add=True)

        @pl.when(jnp.logical_not(even))
        def _():
            gwait(rows1_v, sem1)

            @pl.when(j + 1 < nb)
            def _():
                pltpu.async_copy(h_hbm.at[csrc_v.at[j + 1]], rows0_v, sem0)

            pltpu.sync_copy(rows1_v, acc.at[cdst_v.at[j]], add=True)

        return carry

    lax.fori_loop(0, nb, blk, 0)

    plsc.subcore_barrier()
    pltpu.sync_copy(acc.at[pl.ds(s * RP3, RP3)],
                    out_hbm.at[pl.ds(c * NP3 + s * RP3, RP3)])


_segsum_seed = pl.kernel(
    _seed_body,
    out_type=jax.ShapeDtypeStruct((NC * NP3, CH), jnp.float32),
    mesh=_mesh,
    scratch_types=[
        pltpu.VMEM((EPW,), jnp.int32),
        pltpu.VMEM((EPW,), jnp.int32),
        pltpu.VMEM((SNB, SB), jnp.int32),
        pltpu.VMEM((SNB, SB), jnp.int32),
        pltpu.VMEM((SB, CH), jnp.float32),
        pltpu.VMEM((SB, CH), jnp.float32),
        pltpu.VMEM_SHARED((NP3, CH), jnp.float32),
        pltpu.SemaphoreType.DMA,
        pltpu.SemaphoreType.DMA,
    ],
    compiler_params=pltpu.CompilerParams(use_tc_tiling_on_sc=False,
                                         needs_layout_passes=False),
)


def _make_segsum(C, with_emb=False):
    out_type = jax.ShapeDtypeStruct((NC * NPA, C), jnp.float32)
    scratch = [
        pltpu.VMEM((NBLK, BLK), jnp.int32),
        pltpu.VMEM((NBLK, BLK), jnp.int32),
    ]
    nbuf = 4 if with_emb else 2
    scratch += [pltpu.VMEM((BLK, C), jnp.float32)] * nbuf
    if with_emb:
        out_type = (out_type, jax.ShapeDtypeStruct((NP_EMB, CH), jnp.float32))
        scratch += [
            pltpu.VMEM((EB_NBLK, EB_BLK), jnp.int32),
            pltpu.VMEM((EB_BLK, CH), jnp.float32),
        ]
    scratch += [pltpu.VMEM_SHARED((NPA, C), jnp.float32)]
    scratch += [pltpu.SemaphoreType.DMA] * (2 * nbuf if with_emb else 2)
    return pl.kernel(
        functools.partial(_segsum_body, C, with_emb),
        out_type=out_type,
        mesh=_mesh,
        scratch_types=scratch,
        compiler_params=pltpu.CompilerParams(use_tc_tiling_on_sc=False),
    )


_segsum16_emb = _make_segsum(16, with_emb=True)
_segsum128 = _make_segsum(128)


def _mm(a, b):
    return lax.dot(a, b, precision=lax.Precision.HIGHEST)


def _gin0_body(V_ref, P0a_ref, P0b_ref, W1_ref, b1_ref, W2_ref, b2_ref,
               H1_ref):
    A0 = P0a_ref[...] + P0b_ref[...]
    m = V_ref[...] + A0[:, :K]
    for sgn, col in ((1.0, 0), (-1.0, K * HID)):
        t = jax.nn.relu(_mm(sgn * m, W1_ref[...]) + b1_ref[...])
        t = _mm(t, W2_ref[...]) + b2_ref[...]
        H1_ref[:, col:col + K * HID] = jax.nn.relu(t)


NB = 2000
_grid = (N // NB,)
_rows = lambda c: pl.BlockSpec((NB, c), lambda i: (i, 0))
_full = lambda a, b: pl.BlockSpec((a, b), lambda i: (0, 0))

_tc_gin0 = pl.pallas_call(
    _gin0_body,
    grid=_grid,
    in_specs=[_rows(K), _rows(16), _rows(16),
              _full(K, K * HID), _full(1, K * HID),
              _full(K * HID, K * HID), _full(1, K * HID)],
    out_specs=_rows(CH),
    out_shape=jax.ShapeDtypeStruct((N, CH), jnp.float32),
)


def _pe_body(H1_ref, P1a_ref, P1b_ref, x_ref, emb_ref,
             W1_ref, b1_ref, W2_ref, b2_ref,
             rW1_ref, rb1_ref, rW2_ref, rb2_ref,
             peW_ref, peb_ref, h_ref):
    Mf = H1_ref[...] + P1a_ref[...] + P1b_ref[...]
    phi2 = jax.nn.relu(
        _mm(jax.nn.relu(_mm(Mf, W1_ref[...]) + b1_ref[...]), W2_ref[...]) + b2_ref[...])
    phi = phi2[:, :K * PHI_OUT] + phi2[:, K * PHI_OUT:]
    t = jax.nn.relu(_mm(phi, rW1_ref[...]) + rb1_ref[...])
    PE = _mm(t, rW2_ref[...]) + rb2_ref[...]
    h_ref[...] = x_ref[...] + _mm(PE, peW_ref[...]) + peb_ref[...] + emb_ref[...]


_tc_pe = pl.pallas_call(
    _pe_body,
    grid=_grid,
    in_specs=[_rows(CH), _rows(CH), _rows(CH), _rows(CH), _rows(CH),
              _full(CH, CH), _full(1, CH),
              _full(CH, 2 * K * PHI_OUT), _full(1, 2 * K * PHI_OUT),
              _full(K * PHI_OUT, CH), _full(1, CH),
              _full(CH, PE_DIMS), _full(1, PE_DIMS),
              _full(PE_DIMS, CH), _full(1, CH)],
    out_specs=_rows(CH),
    out_shape=jax.ShapeDtypeStruct((N, CH), jnp.float32),
)


def _sage_body(h_ref, Pa_ref, Pb_ref, P0a_ref, P0b_ref,
               Ws_ref, Wn_ref, b_ref, out_ref):
    dinv = 1.0 / jnp.maximum(P0a_ref[:, K:K + 1] + P0b_ref[:, K:K + 1], 1.0)
    neigh = (Pa_ref[...] + Pb_ref[...]) * dinv
    out_ref[...] = jax.nn.relu(
        _mm(h_ref[...], Ws_ref[...]) + _mm(neigh, Wn_ref[...]) + b_ref[...])


_tc_sage = pl.pallas_call(
    _sage_body,
    grid=_grid,
    in_specs=[_rows(CH), _rows(CH), _rows(CH), _rows(16), _rows(16),
              _full(CH, CH), _full(CH, CH), _full(1, CH)],
    out_specs=_rows(CH),
    out_shape=jax.ShapeDtypeStruct((N, CH), jnp.float32),
)


def _head_body(h_ref, P3a_ref, P3b_ref, P0a_ref, P0b_ref,
               Ws_ref, Wn_ref, b_ref, g_ref, beta_ref, hW_ref, hb_ref,
               out_ref):
    dinv = 1.0 / jnp.maximum(P0a_ref[:, K:K + 1] + P0b_ref[:, K:K + 1], 1.0)
    neigh = (P3a_ref[...] + P3b_ref[...]) * dinv
    h2 = jax.nn.relu(
        _mm(h_ref[...], Ws_ref[...]) + _mm(neigh, Wn_ref[...]) + b_ref[...])
    mu = jnp.mean(h2, axis=-1, keepdims=True)
    var = jnp.mean((h2 - mu) * (h2 - mu), axis=-1, keepdims=True)
    hn = (h2 - mu) * lax.rsqrt(var + 1e-5) * g_ref[...] + beta_ref[...]
    out_ref[...] = _mm(hn, hW_ref[...]) + hb_ref[...]


_tc_head = pl.pallas_call(
    _head_body,
    out_shape=jax.ShapeDtypeStruct((SEED, OUT), jnp.float32),
)


def kernel(x, V, edge_index, n_id, emb_table,
           gin0_W1, gin0_b1, gin0_W2, gin0_b2,
           gin1_W1, gin1_b1, gin1_W2, gin1_b2,
           rho_W1, rho_b1, rho_W2, rho_b2,
           pe_W, pe_b,
           sage0_Wself, sage0_Wneigh, sage0_b,
           sage1_Wself, sage1_Wneigh, sage1_b,
           ln_gamma, ln_beta, head_W, head_b):
    f32 = jnp.float32
    src = edge_index[0].reshape(NW, NBLK, BLK)
    dst = edge_index[1].reshape(NW, NBLK, BLK)

    H0 = jnp.concatenate(
        [V, jnp.ones((N, 1), f32), jnp.zeros((N, 16 - K - 1), f32)], axis=1)
    ids = jnp.concatenate(
        [n_id, jnp.zeros((NP_EMB - N,), jnp.int32)]).reshape(NW, EB_NBLK, EB_BLK)

    P0, EMB = _segsum16_emb(src, dst, H0, ids, emb_table)
    P0a, P0b = P0[:N], P0[NPA:NPA + N]

    eyeK = jnp.eye(K, dtype=f32)
    eye2K = jnp.eye(2 * K, dtype=f32)
    row = lambda v: v.reshape(1, -1)

    H1 = _tc_gin0(V, P0a, P0b,
                  jnp.kron(eyeK, gin0_W1), row(jnp.tile(gin0_b1, K)),
                  jnp.kron(eyeK, gin0_W2), row(jnp.tile(gin0_b2, K)))

    P1 = _segsum128(src, dst, H1)

    h = _tc_pe(H1, P1[:N], P1[NPA:NPA + N], x, EMB[:N],
               jnp.kron(eye2K, gin1_W1), row(jnp.tile(gin1_b1, 2 * K)),
               jnp.kron(eye2K, gin1_W2), row(jnp.tile(gin1_b2, 2 * K)),
               jnp.kron(eyeK, rho_W1), row(jnp.tile(rho_b1, K)),
               jnp.concatenate([rho_W2] * K, axis=0), row(K * rho_b2),
               pe_W, row(pe_b))

    P2 = _segsum128(src, dst, h)
    h = _tc_sage(h, P2[:N], P2[NPA:NPA + N], P0a, P0b,
                 sage0_Wself, sage0_Wneigh, row(sage0_b))

    P3 = _segsum_seed(edge_index[0].reshape(NW, EPW),
                      edge_index[1].reshape(NW, EPW), h)
    out = _tc_head(h[:SEED], P3[:SEED], P3[NP3:NP3 + SEED],
                   P0a[:SEED], P0b[:SEED],
                   sage1_Wself, sage1_Wneigh, row(sage1_b),
                   row(ln_gamma), row(ln_beta), head_W, row(head_b))
    return out


# speedup vs baseline: 136.4355x; 1.0018x over previous
import functools

import jax
import jax.numpy as jnp
from jax import lax
from jax.experimental import pallas as pl
from jax.experimental.pallas import tpu as pltpu
from jax.experimental.pallas import tpu_sc as plsc

N = 10000
E = 320000
K = 4
CH = 128
HID = 16
PHI_OUT = 4
PE_DIMS = 8
VOCAB = 200000
OUT = 64
SEED = 1024

NC = 2
NS = 16
NW = NC * NS
EPW = E // NW
BLK = 100
NBLK = EPW // BLK
NPA = N
RPS = NPA // NS

NP_EMB = 10240
EB_BLK = 80
EB_NBLK = NP_EMB // NW // EB_BLK

_mesh = plsc.VectorSubcoreMesh(core_axis_name="c", subcore_axis_name="s")


def _segsum_body(C, with_emb, *refs):
    if with_emb:
        (src_hbm, dst_hbm, h_hbm, ids_hbm, emb_hbm, out_hbm, embout_hbm,
         src_v, dst_v, rows0_v, rows1_v, rows2_v, rows3_v, eid_v, erow_v, acc,
         sem0, sem1, sem2, sem3, ss0, ss1, ss2, ss3) = refs
        bufs = (rows0_v, rows1_v, rows2_v, rows3_v)
        gsems = (sem0, sem1, sem2, sem3)
        ssems = (ss0, ss1, ss2, ss3)
    else:
        (src_hbm, dst_hbm, h_hbm, out_hbm,
         src_v, dst_v, rows0_v, rows1_v, acc, sem0, sem1) = refs

    c = lax.axis_index("c")
    s = lax.axis_index("s")
    w = c * NS + s

    pltpu.sync_copy(src_hbm.at[w], src_v)
    pltpu.sync_copy(dst_hbm.at[w], dst_v)

    cq = C // 16

    def zrow(i, carry):
        rows0_v[i // cq, pl.ds((i % cq) * 16, 16)] = jnp.zeros((16,), jnp.float32)
        return carry

    lax.fori_loop(0, BLK * cq, zrow, 0)

    def zacc(i, carry):
        pltpu.sync_copy(rows0_v, acc.at[pl.ds(s * RPS + i * BLK, BLK)])
        return carry

    lax.fori_loop(0, RPS // BLK, zacc, 0)
    pltpu.sync_copy(rows0_v.at[pl.ds(0, RPS % BLK)],
                    acc.at[pl.ds(s * RPS + (RPS // BLK) * BLK, RPS % BLK)])
    plsc.subcore_barrier()

    def gather(j, buf, sem):
        pltpu.async_copy(h_hbm.at[src_v.at[j]], buf, sem)

    def gwait(buf, sem):
        pltpu.make_async_copy(h_hbm.at[src_v.at[0]], buf, sem).wait()

    def scat(j, buf):
        pltpu.sync_copy(buf, acc.at[dst_v.at[j]], add=True)

    if with_emb:
        def ascat(j, buf, sem):
            pltpu.async_copy(buf, acc.at[dst_v.at[j]], sem, add=True)

        def sdrain(buf, sem):
            pltpu.make_async_copy(buf, acc.at[dst_v.at[0]], sem).wait()

        gather(0, bufs[0], gsems[0])
        gather(1, bufs[1], gsems[1])

        def edge4(o, carry):
            for b in range(4):
                j = o * 4 + b
                bb = (b + 2) % 4
                gwait(bufs[b], gsems[b])
                ascat(j, bufs[b], ssems[b])

                @pl.when(j + 2 < NBLK)
                def _():
                    @pl.when(j >= 2)
                    def _():
                        sdrain(bufs[bb], ssems[bb])
                    gather(j + 2, bufs[bb], gsems[bb])
            return carry

        lax.fori_loop(0, NBLK // 4, edge4, 0)
        for b in range(4):
            sdrain(bufs[b], ssems[b])
    else:
        gather(0, rows0_v, sem0)
        gather(1, rows1_v, sem1)

        def edge(i, carry):
            j = i * 2
            gwait(rows0_v, sem0)
            scat(j, rows0_v)
            gather(j + 2, rows0_v, sem0)
            gwait(rows1_v, sem1)
            scat(j + 1, rows1_v)
            gather(j + 3, rows1_v, sem1)
            return carry

        lax.fori_loop(0, NBLK // 2 - 1, edge, 0)
        gwait(rows0_v, sem0)
        scat(NBLK - 2, rows0_v)
        gwait(rows1_v, sem1)
        scat(NBLK - 1, rows1_v)

    if with_emb:
        pltpu.sync_copy(ids_hbm.at[w], eid_v)

        def eblk(t, carry):
            pltpu.async_copy(emb_hbm.at[eid_v.at[t]], erow_v, sem0).wait()
            pltpu.sync_copy(
                erow_v, embout_hbm.at[pl.ds(w * (EB_NBLK * EB_BLK) + t * EB_BLK,
                                            EB_BLK)])
            return carry

        lax.fori_loop(0, EB_NBLK, eblk, 0)

    plsc.subcore_barrier()
    pltpu.sync_copy(acc.at[pl.ds(s * RPS, RPS)],
                    out_hbm.at[pl.ds(c * NPA + s * RPS, RPS)])


NP3 = 1040
RP3 = NP3 // NS
SB = 128
SNB = 79


def _seed_body(src_hbm, dst_hbm, h_hbm, out_hbm,
               srcf_v, dstf_v, csrc_v, cdst_v, rows0_v, rows1_v, acc,
               sem0, sem1):
    c = lax.axis_index("c")
    s = lax.axis_index("s")
    w = c * NS + s

    pltpu.sync_copy(src_hbm.at[w], srcf_v)
    pltpu.sync_copy(dst_hbm.at[w], dstf_v)

    zero16 = jnp.zeros((16,), jnp.int32)
    dummy16 = jnp.full((16,), SEED, jnp.int32)

    @plsc.parallel_loop(0, SNB * 8, unroll=4)
    def _(i):
        csrc_v[i // 8, pl.ds((i % 8) * 16, 16)] = zero16
        cdst_v[i // 8, pl.ds((i % 8) * 16, 16)] = dummy16

    @plsc.parallel_loop(0, EPW // 16, unroll=4,
                        carry=jnp.zeros((16,), jnp.int32))
    def base(i, b):
        d16 = dstf_v[pl.ds(i * 16, 16)]
        s16 = srcf_v[pl.ds(i * 16, 16)]
        m = d16 < SEED
        mi = jnp.where(m, 1, 0).astype(jnp.int32)
        pos = b + plsc.cumsum(mi) - mi
        pdiv = lax.shift_right_logical(pos, 7)
        pmod = lax.bitwise_and(pos, 127)
        plsc.store_scatter(csrc_v, [pdiv, pmod], s16, mask=m)
        plsc.store_scatter(cdst_v, [pdiv, pmod], d16, mask=m)
        return b + plsc.all_reduce_population_count(m)
    cnt = lax.reduce_max(base, axes=(0,))
    nb = (cnt + (SB - 1)) // SB

    def zrow(i, carry):
        rows0_v[i // 8, pl.ds((i % 8) * 16, 16)] = jnp.zeros((16,), jnp.float32)
        return carry

    lax.fori_loop(0, RP3 * 8, zrow, 0)
    pltpu.sync_copy(rows0_v.at[pl.ds(0, RP3)], acc.at[pl.ds(s * RP3, RP3)])
    plsc.subcore_barrier()

    def gwait(buf, sem):
        pltpu.make_async_copy(h_hbm.at[csrc_v.at[0]], buf, sem).wait()

    @pl.when(nb > 0)
    def _():
        pltpu.async_copy(h_hbm.at[csrc_v.at[0]], rows0_v, sem0)

    def blk(j, carry):
        even = lax.rem(j, 2) == 0

        @pl.when(even)
        def _():
            gwait(rows0_v, sem0)

            @pl.when(j + 1 < nb)
            def _():
                pltpu.async_copy(h_hbm.at[csrc_v.at[j + 1]], rows1_v, sem1)

            pltpu.sync_copy(rows0_v, acc.at[cdst_v.at[j]], add=True)

        @pl.when(jnp.logical_not(even))
        def _():
            gwait(rows1_v, sem1)

            @pl.when(j + 1 < nb)
            def _():
                pltpu.async_copy(h_hbm.at[csrc_v.at[j + 1]], rows0_v, sem0)

            pltpu.sync_copy(rows1_v, acc.at[cdst_v.at[j]], add=True)

        return carry

    lax.fori_loop(0, nb, blk, 0)

    plsc.subcore_barrier()
    pltpu.sync_copy(acc.at[pl.ds(s * RP3, RP3)],
                    out_hbm.at[pl.ds(c * NP3 + s * RP3, RP3)])


_segsum_seed = pl.kernel(
    _seed_body,
    out_type=jax.ShapeDtypeStruct((NC * NP3, CH), jnp.float32),
    mesh=_mesh,
    scratch_types=[
        pltpu.VMEM((EPW,), jnp.int32),
        pltpu.VMEM((EPW,), jnp.int32),
        pltpu.VMEM((SNB, SB), jnp.int32),
        pltpu.VMEM((SNB, SB), jnp.int32),
        pltpu.VMEM((SB, CH), jnp.float32),
        pltpu.VMEM((SB, CH), jnp.float32),
        pltpu.VMEM_SHARED((NP3, CH), jnp.float32),
        pltpu.SemaphoreType.DMA,
        pltpu.SemaphoreType.DMA,
    ],
    compiler_params=pltpu.CompilerParams(use_tc_tiling_on_sc=False,
                                         needs_layout_passes=False),
)


def _make_segsum(C, with_emb=False):
    out_type = jax.ShapeDtypeStruct((NC * NPA, C), jnp.float32)
    scratch = [
        pltpu.VMEM((NBLK, BLK), jnp.int32),
        pltpu.VMEM((NBLK, BLK), jnp.int32),
    ]
    nbuf = 4 if with_emb else 2
    scratch += [pltpu.VMEM((BLK, C), jnp.float32)] * nbuf
    if with_emb:
        out_type = (out_type, jax.ShapeDtypeStruct((NP_EMB, CH), jnp.float32))
        scratch += [
            pltpu.VMEM((EB_NBLK, EB_BLK), jnp.int32),
            pltpu.VMEM((EB_BLK, CH), jnp.float32),
        ]
    scratch += [pltpu.VMEM_SHARED((NPA, C), jnp.float32)]
    scratch += [pltpu.SemaphoreType.DMA] * (2 * nbuf if with_emb else 2)
    return pl.kernel(
        functools.partial(_segsum_body, C, with_emb),
        out_type=out_type,
        mesh=_mesh,
        scratch_types=scratch,
        compiler_params=pltpu.CompilerParams(use_tc_tiling_on_sc=False),
    )


_segsum16_emb = _make_segsum(16, with_emb=True)
_segsum128 = _make_segsum(128)


def _mm(a, b):
    return lax.dot(a, b, precision=lax.Precision.HIGHEST)


def _gin0_body(V_ref, P0a_ref, P0b_ref, W1_ref, b1_ref, W2_ref, b2_ref,
               H1_ref):
    A0 = P0a_ref[...] + P0b_ref[...]
    m = V_ref[...] + A0[:, :K]
    for sgn, col in ((1.0, 0), (-1.0, K * HID)):
        t = jax.nn.relu(_mm(sgn * m, W1_ref[...]) + b1_ref[...])
        t = _mm(t, W2_ref[...]) + b2_ref[...]
        H1_ref[:, col:col + K * HID] = jax.nn.relu(t)


NB = 2000
_grid = (N // NB,)
_rows = lambda c: pl.BlockSpec((NB, c), lambda i: (i, 0))
_full = lambda a, b: pl.BlockSpec((a, b), lambda i: (0, 0))

_tc_gin0 = pl.pallas_call(
    _gin0_body,
    grid=_grid,
    in_specs=[_rows(K), _rows(16), _rows(16),
              _full(K, K * HID), _full(1, K * HID),
              _full(K * HID, K * HID), _full(1, K * HID)],
    out_specs=_rows(CH),
    out_shape=jax.ShapeDtypeStruct((N, CH), jnp.float32),
)


def _pe_body(H1_ref, P1a_ref, P1b_ref, x_ref, emb_ref,
             W1_ref, b1_ref, W2_ref, b2_ref,
             rW1_ref, rb1_ref, rW2_ref, rb2_ref,
             peW_ref, peb_ref, h_ref):
    Mf = H1_ref[...] + P1a_ref[...] + P1b_ref[...]
    phi2 = jax.nn.relu(
        _mm(jax.nn.relu(_mm(Mf, W1_ref[...]) + b1_ref[...]), W2_ref[...]) + b2_ref[...])
    phi = phi2[:, :K * PHI_OUT] + phi2[:, K * PHI_OUT:]
    t = jax.nn.relu(_mm(phi, rW1_ref[...]) + rb1_ref[...])
    PE = _mm(t, rW2_ref[...]) + rb2_ref[...]
    h_ref[...] = x_ref[...] + _mm(PE, peW_ref[...]) + peb_ref[...] + emb_ref[...]


_tc_pe = pl.pallas_call(
    _pe_body,
    grid=_grid,
    in_specs=[_rows(CH), _rows(CH), _rows(CH), _rows(CH), _rows(CH),
              _full(CH, CH), _full(1, CH),
              _full(CH, 2 * K * PHI_OUT), _full(1, 2 * K * PHI_OUT),
              _full(K * PHI_OUT, CH), _full(1, CH),
              _full(CH, PE_DIMS), _full(1, PE_DIMS),
              _full(PE_DIMS, CH), _full(1, CH)],
    out_specs=_rows(CH),
    out_shape=jax.ShapeDtypeStruct((N, CH), jnp.float32),
)


def _sage_body(h_ref, Pa_ref, Pb_ref, P0a_ref, P0b_ref,
               Ws_ref, Wn_ref, b_ref, out_ref):
    dinv = 1.0 / jnp.maximum(P0a_ref[:, K:K + 1] + P0b_ref[:, K:K + 1], 1.0)
    neigh = (Pa_ref[...] + Pb_ref[...]) * dinv
    out_ref[...] = jax.nn.relu(
        _mm(h_ref[...], Ws_ref[...]) + _mm(neigh, Wn_ref[...]) + b_ref[...])


_tc_sage = pl.pallas_call(
    _sage_body,
    grid=_grid,
    in_specs=[_rows(CH), _rows(CH), _rows(CH), _rows(16), _rows(16),
              _full(CH, CH), _full(CH, CH), _full(1, CH)],
    out_specs=_rows(CH),
    out_shape=jax.ShapeDtypeStruct((N, CH), jnp.float32),
)


def _head_body(h_ref, P3a_ref, P3b_ref, P0a_ref, P0b_ref,
               Ws_ref, Wn_ref, b_ref, g_ref, beta_ref, hW_ref, hb_ref,
               out_ref):
    dinv = 1.0 / jnp.maximum(P0a_ref[:, K:K + 1] + P0b_ref[:, K:K + 1], 1.0)
    neigh = (P3a_ref[...] + P3b_ref[...]) * dinv
    h2 = jax.nn.relu(
        _mm(h_ref[...], Ws_ref[...]) + _mm(neigh, Wn_ref[...]) + b_ref[...])
    mu = jnp.mean(h2, axis=-1, keepdims=True)
    var = jnp.mean((h2 - mu) * (h2 - mu), axis=-1, keepdims=True)
    hn = (h2 - mu) * lax.rsqrt(var + 1e-5) * g_ref[...] + beta_ref[...]
    out_ref[...] = _mm(hn, hW_ref[...]) + hb_ref[...]


_tc_head = pl.pallas_call(
    _head_body,
    out_shape=jax.ShapeDtypeStruct((SEED, OUT), jnp.float32),
)


def kernel(x, V, edge_index, n_id, emb_table,
           gin0_W1, gin0_b1, gin0_W2, gin0_b2,
           gin1_W1, gin1_b1, gin1_W2, gin1_b2,
           rho_W1, rho_b1, rho_W2, rho_b2,
           pe_W, pe_b,
           sage0_Wself, sage0_Wneigh, sage0_b,
           sage1_Wself, sage1_Wneigh, sage1_b,
           ln_gamma, ln_beta, head_W, head_b):
    f32 = jnp.float32
    src = edge_index[0].reshape(NW, NBLK, BLK)
    dst = edge_index[1].reshape(NW, NBLK, BLK)

    H0 = jnp.concatenate(
        [V, jnp.ones((N, 1), f32), jnp.zeros((N, 16 - K - 1), f32)], axis=1)
    ids = jnp.concatenate(
        [n_id, jnp.zeros((NP_EMB - N,), jnp.int32)]).reshape(NW, EB_NBLK, EB_BLK)

    eyeK = jnp.eye(K, dtype=f32)
    eye2K = jnp.eye(2 * K, dtype=f32)
    row = lambda v: v.reshape(1, -1)
    g0W1, g0b1 = jnp.kron(eyeK, gin0_W1), row(jnp.tile(gin0_b1, K))
    g0W2, g0b2 = jnp.kron(eyeK, gin0_W2), row(jnp.tile(gin0_b2, K))
    g1W1, g1b1 = jnp.kron(eye2K, gin1_W1), row(jnp.tile(gin1_b1, 2 * K))
    g1W2, g1b2 = jnp.kron(eye2K, gin1_W2), row(jnp.tile(gin1_b2, 2 * K))
    rW1, rb1 = jnp.kron(eyeK, rho_W1), row(jnp.tile(rho_b1, K))
    rW2, rb2 = jnp.concatenate([rho_W2] * K, axis=0), row(K * rho_b2)

    P0, EMB = _segsum16_emb(src, dst, H0, ids, emb_table)
    P0a, P0b = P0[:N], P0[NPA:NPA + N]

    H1 = _tc_gin0(V, P0a, P0b, g0W1, g0b1, g0W2, g0b2)

    P1 = _segsum128(src, dst, H1)

    h = _tc_pe(H1, P1[:N], P1[NPA:NPA + N], x, EMB[:N],
               g1W1, g1b1, g1W2, g1b2, rW1, rb1, rW2, rb2,
               pe_W, row(pe_b))

    P2 = _segsum128(src, dst, h)
    h = _tc_sage(h, P2[:N], P2[NPA:NPA + N], P0a, P0b,
                 sage0_Wself, sage0_Wneigh, row(sage0_b))

    P3 = _segsum_seed(edge_index[0].reshape(NW, EPW),
                      edge_index[1].reshape(NW, EPW), h)
    out = _tc_head(h[:SEED], P3[:SEED], P3[NP3:NP3 + SEED],
                   P0a[:SEED], P0b[:SEED],
                   sage1_Wself, sage1_Wneigh, row(sage1_b),
                   row(ln_gamma), row(ln_beta), head_W, row(head_b))
    return out
